# baseline (speedup 1.0000x reference)
import functools

import jax
import jax.numpy as jnp
from jax import lax
from jax.experimental import pallas as pl
from jax.experimental.pallas import tpu as pltpu
from jax.experimental.pallas import tpu_sc as plsc

N_NODES = 10000
N_EDGES = 160000
NUM_RBF = 64
EDGE_CH = 64
NUM_COEFF = 16
M0C = 4
SPH = 64
MAX_ELEM = 90
HIDDEN = 128
RESCALE = 16.0
OUT_CH = NUM_COEFF * SPH
FEATC = NUM_RBF + NUM_COEFF * M0C

EB = 640
NCH = N_EDGES // EB
WIN = 128
NPAD = 10112

_NC = 2
_NS = 16
_NW = _NC * _NS
_CK = 128
_ROWS_PT = 40
_PER_TILE = _CK * _ROWS_PT
E_PAD = _NW * _PER_TILE
_IDX_ROWS = E_PAD // _CK


def _sc_body(feat_hbm, ns_hbm, nt_hbm, perm_hbm, srcp_hbm, dsts_hbm,
             feat_out, gsrc_out, gdst_out,
             ixf, ixs, ixd, fa, sa, da, fb, sb, db,
             s1, s2, s3, s4, s5, s6):
    wid = lax.axis_index("s") * _NC + lax.axis_index("c")
    base = wid * _PER_TILE
    rbase = wid * _ROWS_PT
    pltpu.sync_copy(perm_hbm.at[pl.ds(rbase, _ROWS_PT)], ixf)
    pltpu.sync_copy(srcp_hbm.at[pl.ds(rbase, _ROWS_PT)], ixs)
    pltpu.sync_copy(dsts_hbm.at[pl.ds(rbase, _ROWS_PT)], ixd)

    def pair(g, carry):
        r0 = 2 * g
        r1 = r0 + 1
        off_a = base + r0 * _CK
        off_b = base + r1 * _CK
        a_f = pltpu.async_copy(feat_hbm.at[ixf.at[r0]], fa, s1)
        a_s = pltpu.async_copy(ns_hbm.at[ixs.at[r0]], sa, s2)
        a_d = pltpu.async_copy(nt_hbm.at[ixd.at[r0]], da, s3)
        b_f = pltpu.async_copy(feat_hbm.at[ixf.at[r1]], fb, s4)
        b_s = pltpu.async_copy(ns_hbm.at[ixs.at[r1]], sb, s5)
        b_d = pltpu.async_copy(nt_hbm.at[ixd.at[r1]], db, s6)
        a_f.wait()
        pltpu.sync_copy(fa, feat_out.at[pl.ds(off_a, _CK)])
        a_s.wait()
        pltpu.sync_copy(sa, gsrc_out.at[pl.ds(off_a, _CK)])
        a_d.wait()
        pltpu.sync_copy(da, gdst_out.at[pl.ds(off_a, _CK)])
        b_f.wait()
        pltpu.sync_copy(fb, feat_out.at[pl.ds(off_b, _CK)])
        b_s.wait()
        pltpu.sync_copy(sb, gsrc_out.at[pl.ds(off_b, _CK)])
        b_d.wait()
        pltpu.sync_copy(db, gdst_out.at[pl.ds(off_b, _CK)])
        return carry

    lax.fori_loop(0, _ROWS_PT // 2, pair, 0)


def _sc_gather(feat, ns, nt, perm, src_p, dst_s):
    mesh = plsc.VectorSubcoreMesh(core_axis_name="c", subcore_axis_name="s")
    f = functools.partial(
        pl.kernel, mesh=mesh,
        out_type=[
            jax.ShapeDtypeStruct((E_PAD, FEATC), jnp.float32),
            jax.ShapeDtypeStruct((E_PAD, HIDDEN), jnp.float32),
            jax.ShapeDtypeStruct((E_PAD, HIDDEN), jnp.float32),
        ],
        scratch_types=[
            pltpu.VMEM((_ROWS_PT, _CK), jnp.int32),
            pltpu.VMEM((_ROWS_PT, _CK), jnp.int32),
            pltpu.VMEM((_ROWS_PT, _CK), jnp.int32),
            pltpu.VMEM((_CK, FEATC), jnp.float32),
            pltpu.VMEM((_CK, HIDDEN), jnp.float32),
            pltpu.VMEM((_CK, HIDDEN), jnp.float32),
            pltpu.VMEM((_CK, FEATC), jnp.float32),
            pltpu.VMEM((_CK, HIDDEN), jnp.float32),
            pltpu.VMEM((_CK, HIDDEN), jnp.float32),
            pltpu.SemaphoreType.DMA,
            pltpu.SemaphoreType.DMA,
            pltpu.SemaphoreType.DMA,
            pltpu.SemaphoreType.DMA,
            pltpu.SemaphoreType.DMA,
            pltpu.SemaphoreType.DMA,
        ],
    )
    pad = ((0, E_PAD - N_EDGES),)
    perm2 = jnp.pad(perm, pad).reshape(_IDX_ROWS, _CK)
    srcp2 = jnp.pad(src_p, pad).reshape(_IDX_ROWS, _CK)
    dsts2 = jnp.pad(dst_s, pad).reshape(_IDX_ROWS, _CK)
    return f(_sc_body)(feat, ns, nt, perm2, srcp2, dsts2)


def _tc_body(win_lo_ref, n_win_ref, dst_col_ref,
             feat_ref, gsrc_ref, gdst_ref, w1d_ref, w2_ref,
             b2_ref, out_ref, coeff_ref):
    b = pl.program_id(0)

    @pl.when(b == 0)
    def _init():
        out_ref[...] = jnp.zeros_like(out_ref)

    gsum = gsrc_ref[...] + gdst_ref[...]
    r128 = jax.lax.broadcasted_iota(jnp.int32, (HIDDEN, HIDDEN), 0)
    c128 = jax.lax.broadcasted_iota(jnp.int32, (HIDDEN, HIDDEN), 1)
    eye128 = (r128 == c128).astype(jnp.float32)
    dist = feat_ref[:, :NUM_RBF]
    c11 = (((1,), (1,)), ((), ()))
    x = lax.dot_general(w1d_ref[...], dist, c11,
                        preferred_element_type=jnp.float32)
    x = x + lax.dot_general(eye128, gsum, c11,
                            preferred_element_type=jnp.float32)
    h = jnp.maximum(x, 0.0)
    m0 = jnp.dot(w2_ref[...], h, preferred_element_type=jnp.float32)
    m0 = (m0 + b2_ref[...]).astype(jnp.bfloat16)

    eye64 = eye128[:SPH, :SPH]
    wig = lax.dot_general(eye64, feat_ref[:, NUM_RBF:], c11,
                          preferred_element_type=jnp.float32
                          ).astype(jnp.bfloat16)
    for j in range(M0C):
        m0j = m0[j * SPH:(j + 1) * SPH, :]
        for i in range(NUM_COEFF):
            r = i * M0C + j
            w_row = lax.broadcast_in_dim(wig[r:r + 1, :], (SPH, EB), (0, 1))
            contrib = w_row * m0j
            sl = slice(i * SPH, (i + 1) * SPH)
            if j == 0:
                coeff_ref[sl, :] = contrib
            else:
                coeff_ref[sl, :] += contrib

    dst_col = dst_col_ref[0]
    win_lo = win_lo_ref[b]
    n_win = n_win_ref[b]
    coeff = coeff_ref[...]

    def body(w, carry):
        base = (win_lo + w) * WIN
        lane = jax.lax.broadcasted_iota(jnp.int32, (EB, WIN), 1).astype(jnp.float32)
        oh = (lane == (dst_col - base.astype(jnp.float32))).astype(jnp.bfloat16)
        contrib = jnp.dot(coeff, oh, preferred_element_type=jnp.float32)
        out_ref[:, pl.ds(base, WIN)] += contrib
        return carry

    lax.fori_loop(0, n_win, body, 0)


def kernel(atomic_numbers, edge_distance, edge_index, wigner_inv,
           source_embedding, target_embedding, W1, b1, W2, b2):
    src = edge_index[0]
    dst = edge_index[1]
    eid = jnp.arange(N_EDGES, dtype=jnp.uint32)
    key = dst.astype(jnp.uint32) * jnp.uint32(1 << 18) + eid
    val = src.astype(jnp.uint32) * jnp.uint32(1 << 18) + eid
    key_s, val_s = lax.sort([key, val], num_keys=1)
    perm = (val_s & jnp.uint32((1 << 18) - 1)).astype(jnp.int32)
    src_p = (val_s >> 18).astype(jnp.int32)
    dst_s = (key_s >> 18).astype(jnp.int32)

    feat = jnp.concatenate(
        [edge_distance, wigner_inv.reshape(N_EDGES, NUM_COEFF * M0C)], axis=1)

    oh_elem = (atomic_numbers[:, None] ==
               jnp.arange(MAX_ELEM)[None, :]).astype(jnp.float32)
    ns = oh_elem @ (source_embedding @ W1[NUM_RBF:NUM_RBF + EDGE_CH]) + 0.5 * b1
    nt = oh_elem @ (target_embedding @ W1[NUM_RBF + EDGE_CH:]) + 0.5 * b1

    feat_srt, gsrc, gdst = _sc_gather(feat, ns, nt, perm, src_p, dst_s)

    w1d_t = W1[:NUM_RBF].T
    w2_t = (W2 / RESCALE).T
    b2_c = (b2 / RESCALE).reshape(-1, 1)

    win_lo = (dst_s[::EB] // WIN).astype(jnp.int32)
    win_hi = (dst_s[EB - 1::EB] // WIN).astype(jnp.int32)
    n_win = win_hi - win_lo + 1

    dst_col = dst_s.astype(jnp.float32).reshape(NCH, EB, 1)

    grid_spec = pltpu.PrefetchScalarGridSpec(
        num_scalar_prefetch=2,
        grid=(NCH,),
        in_specs=[
            pl.BlockSpec((1, EB, 1), lambda b, *_: (b, 0, 0)),
            pl.BlockSpec((EB, FEATC), lambda b, *_: (b, 0)),
            pl.BlockSpec((EB, HIDDEN), lambda b, *_: (b, 0)),
            pl.BlockSpec((EB, HIDDEN), lambda b, *_: (b, 0)),
            pl.BlockSpec((HIDDEN, NUM_RBF), lambda b, *_: (0, 0)),
            pl.BlockSpec((M0C * SPH, HIDDEN), lambda b, *_: (0, 0)),
            pl.BlockSpec((M0C * SPH, 1), lambda b, *_: (0, 0)),
        ],
        out_specs=pl.BlockSpec((OUT_CH, NPAD), lambda b, *_: (0, 0)),
        scratch_shapes=[pltpu.VMEM((OUT_CH, EB), jnp.bfloat16)],
    )
    out = pl.pallas_call(
        _tc_body,
        grid_spec=grid_spec,
        out_shape=jax.ShapeDtypeStruct((OUT_CH, NPAD), jnp.float32),
        compiler_params=pltpu.CompilerParams(
            dimension_semantics=("arbitrary",)),
    )(win_lo, n_win, dst_col, feat_srt, gsrc, gdst, w1d_t, w2_t, b2_c)
    return out[:, :N_NODES].T.reshape(N_NODES, NUM_COEFF, SPH)


# speedup vs baseline: 1.1028x; 1.1028x over previous
import functools

import jax
import jax.numpy as jnp
from jax import lax
from jax.experimental import pallas as pl
from jax.experimental.pallas import tpu as pltpu
from jax.experimental.pallas import tpu_sc as plsc

N_NODES = 10000
N_EDGES = 160000
NUM_RBF = 64
EDGE_CH = 64
NUM_COEFF = 16
M0C = 4
SPH = 64
MAX_ELEM = 90
HIDDEN = 128
RESCALE = 16.0
OUT_CH = NUM_COEFF * SPH
FEATC = NUM_RBF + NUM_COEFF * M0C

EB = 640
NCH = N_EDGES // EB
WIN = 128
NPAD = 10112

_NC = 2
_NS = 16
_NW = _NC * _NS
_PER_TILE = N_EDGES // _NW
_CK = 128
_NFULL = _PER_TILE // _CK
_TAIL = _PER_TILE - _NFULL * _CK


def _sc_body(feat_hbm, ns_hbm, nt_hbm, perm_hbm, srcp_hbm, dsts_hbm,
             feat_out, gsrc_out, gdst_out,
             i1, i2, i3, fb, sb, db, i1t, i2t, i3t, fbt, sbt, dbt,
             sem1, sem2, sem3):
    wid = lax.axis_index("s") * _NC + lax.axis_index("c")
    base = wid * _PER_TILE

    def chunk(off, iv1, iv2, iv3, fbuf, sbuf, dbuf, k):
        pltpu.sync_copy(perm_hbm.at[pl.ds(off, k)], iv1)
        pltpu.sync_copy(srcp_hbm.at[pl.ds(off, k)], iv2)
        pltpu.sync_copy(dsts_hbm.at[pl.ds(off, k)], iv3)
        a = pltpu.async_copy(feat_hbm.at[iv1], fbuf, sem1)
        b = pltpu.async_copy(ns_hbm.at[iv2], sbuf, sem2)
        c = pltpu.async_copy(nt_hbm.at[iv3], dbuf, sem3)
        a.wait()
        pltpu.sync_copy(fbuf, feat_out.at[pl.ds(off, k)])
        b.wait()
        pltpu.sync_copy(sbuf, gsrc_out.at[pl.ds(off, k)])
        c.wait()
        pltpu.sync_copy(dbuf, gdst_out.at[pl.ds(off, k)])

    def body(i, carry):
        chunk(base + i * _CK, i1, i2, i3, fb, sb, db, _CK)
        return carry

    lax.fori_loop(0, _NFULL, body, 0)
    chunk(base + _NFULL * _CK, i1t, i2t, i3t, fbt, sbt, dbt, _TAIL)


def _sc_gather(feat, ns, nt, perm, src_p, dst_s):
    mesh = plsc.VectorSubcoreMesh(core_axis_name="c", subcore_axis_name="s")
    f = functools.partial(
        pl.kernel, mesh=mesh,
        out_type=[
            jax.ShapeDtypeStruct((N_EDGES, FEATC), jnp.float32),
            jax.ShapeDtypeStruct((N_EDGES, HIDDEN), jnp.float32),
            jax.ShapeDtypeStruct((N_EDGES, HIDDEN), jnp.float32),
        ],
        scratch_types=[
            pltpu.VMEM((_CK,), jnp.int32),
            pltpu.VMEM((_CK,), jnp.int32),
            pltpu.VMEM((_CK,), jnp.int32),
            pltpu.VMEM((_CK, FEATC), jnp.float32),
            pltpu.VMEM((_CK, HIDDEN), jnp.float32),
            pltpu.VMEM((_CK, HIDDEN), jnp.float32),
            pltpu.VMEM((_TAIL,), jnp.int32),
            pltpu.VMEM((_TAIL,), jnp.int32),
            pltpu.VMEM((_TAIL,), jnp.int32),
            pltpu.VMEM((_TAIL, FEATC), jnp.float32),
            pltpu.VMEM((_TAIL, HIDDEN), jnp.float32),
            pltpu.VMEM((_TAIL, HIDDEN), jnp.float32),
            pltpu.SemaphoreType.DMA,
            pltpu.SemaphoreType.DMA,
            pltpu.SemaphoreType.DMA,
        ],
    )
    return f(_sc_body)(feat, ns, nt, perm, src_p, dst_s)


def _tc_body(win_lo_ref, n_win_ref, dst_col_ref,
             feat_ref, gsrc_ref, gdst_ref, w1d_ref, w2_ref,
             b2_ref, out_ref, coeff_ref):
    b = pl.program_id(0)

    @pl.when(b == 0)
    def _init():
        out_ref[...] = jnp.zeros_like(out_ref)

    gsum = (gsrc_ref[...] + gdst_ref[...]).astype(jnp.bfloat16)
    r128 = jax.lax.broadcasted_iota(jnp.int32, (HIDDEN, HIDDEN), 0)
    c128 = jax.lax.broadcasted_iota(jnp.int32, (HIDDEN, HIDDEN), 1)
    eye128 = (r128 == c128).astype(jnp.bfloat16)
    dist = feat_ref[:, :NUM_RBF].astype(jnp.bfloat16)
    c11 = (((1,), (1,)), ((), ()))
    x = lax.dot_general(w1d_ref[...], dist, c11,
                        preferred_element_type=jnp.float32)
    x = x + lax.dot_general(eye128, gsum, c11,
                            preferred_element_type=jnp.float32)
    h = jnp.maximum(x, 0.0).astype(jnp.bfloat16)
    m0 = jnp.dot(w2_ref[...], h, preferred_element_type=jnp.float32)
    m0 = (m0 + b2_ref[...]).astype(jnp.bfloat16)

    eye64 = eye128[:SPH, :SPH]
    wig = lax.dot_general(eye64, feat_ref[:, NUM_RBF:].astype(jnp.bfloat16),
                          c11, preferred_element_type=jnp.float32
                          ).astype(jnp.bfloat16)
    for j in range(M0C):
        m0j = m0[j * SPH:(j + 1) * SPH, :]
        for i in range(NUM_COEFF):
            r = i * M0C + j
            w_row = lax.broadcast_in_dim(wig[r:r + 1, :], (SPH, EB), (0, 1))
            contrib = w_row * m0j
            sl = slice(i * SPH, (i + 1) * SPH)
            if j == 0:
                coeff_ref[sl, :] = contrib
            else:
                coeff_ref[sl, :] += contrib

    dst_col = dst_col_ref[0]
    win_lo = win_lo_ref[b]
    n_win = n_win_ref[b]
    coeff = coeff_ref[...]

    def body(w, carry):
        base = (win_lo + w) * WIN
        lane = jax.lax.broadcasted_iota(jnp.int32, (EB, WIN), 1).astype(jnp.float32)
        oh = (lane == (dst_col - base.astype(jnp.float32))).astype(jnp.bfloat16)
        contrib = jnp.dot(coeff, oh, preferred_element_type=jnp.float32)
        out_ref[:, pl.ds(base, WIN)] += contrib
        return carry

    lax.fori_loop(0, n_win, body, 0)


def kernel(atomic_numbers, edge_distance, edge_index, wigner_inv,
           source_embedding, target_embedding, W1, b1, W2, b2):
    src = edge_index[0]
    dst = edge_index[1]
    eid = jnp.arange(N_EDGES, dtype=jnp.uint32)
    key = dst.astype(jnp.uint32) * jnp.uint32(1 << 18) + eid
    val = src.astype(jnp.uint32) * jnp.uint32(1 << 18) + eid
    key_s, val_s = lax.sort([key, val], num_keys=1)
    perm = (val_s & jnp.uint32((1 << 18) - 1)).astype(jnp.int32)
    src_p = (val_s >> 18).astype(jnp.int32)
    dst_s = (key_s >> 18).astype(jnp.int32)

    feat = jnp.concatenate(
        [edge_distance, wigner_inv.reshape(N_EDGES, NUM_COEFF * M0C)], axis=1)

    oh_elem = (atomic_numbers[:, None] ==
               jnp.arange(MAX_ELEM)[None, :]).astype(jnp.float32)
    ns = oh_elem @ (source_embedding @ W1[NUM_RBF:NUM_RBF + EDGE_CH]) + 0.5 * b1
    nt = oh_elem @ (target_embedding @ W1[NUM_RBF + EDGE_CH:]) + 0.5 * b1

    feat_srt, gsrc, gdst = _sc_gather(feat, ns, nt, perm, src_p, dst_s)

    w1d_t = W1[:NUM_RBF].T.astype(jnp.bfloat16)
    w2_t = (W2 / RESCALE).T.astype(jnp.bfloat16)
    b2_c = (b2 / RESCALE).reshape(-1, 1)

    win_lo = (dst_s[::EB] // WIN).astype(jnp.int32)
    win_hi = (dst_s[EB - 1::EB] // WIN).astype(jnp.int32)
    n_win = win_hi - win_lo + 1

    dst_col = dst_s.astype(jnp.float32).reshape(NCH, EB, 1)

    grid_spec = pltpu.PrefetchScalarGridSpec(
        num_scalar_prefetch=2,
        grid=(NCH,),
        in_specs=[
            pl.BlockSpec((1, EB, 1), lambda b, *_: (b, 0, 0)),
            pl.BlockSpec((EB, FEATC), lambda b, *_: (b, 0)),
            pl.BlockSpec((EB, HIDDEN), lambda b, *_: (b, 0)),
            pl.BlockSpec((EB, HIDDEN), lambda b, *_: (b, 0)),
            pl.BlockSpec((HIDDEN, NUM_RBF), lambda b, *_: (0, 0)),
            pl.BlockSpec((M0C * SPH, HIDDEN), lambda b, *_: (0, 0)),
            pl.BlockSpec((M0C * SPH, 1), lambda b, *_: (0, 0)),
        ],
        out_specs=pl.BlockSpec((OUT_CH, NPAD), lambda b, *_: (0, 0)),
        scratch_shapes=[pltpu.VMEM((OUT_CH, EB), jnp.bfloat16)],
    )
    out = pl.pallas_call(
        _tc_body,
        grid_spec=grid_spec,
        out_shape=jax.ShapeDtypeStruct((OUT_CH, NPAD), jnp.float32),
        compiler_params=pltpu.CompilerParams(
            dimension_semantics=("arbitrary",)),
    )(win_lo, n_win, dst_col, feat_srt, gsrc, gdst, w1d_t, w2_t, b2_c)
    return out[:, :N_NODES].T.reshape(N_NODES, NUM_COEFF, SPH)


# speedup vs baseline: 1.1630x; 1.0545x over previous
import functools

import jax
import jax.numpy as jnp
from jax import lax
from jax.experimental import pallas as pl
from jax.experimental.pallas import tpu as pltpu
from jax.experimental.pallas import tpu_sc as plsc

N_NODES = 10000
N_EDGES = 160000
NUM_RBF = 64
EDGE_CH = 64
NUM_COEFF = 16
M0C = 4
SPH = 64
MAX_ELEM = 90
HIDDEN = 128
RESCALE = 16.0
OUT_CH = NUM_COEFF * SPH
FEATC = NUM_RBF + NUM_COEFF * M0C

EB = 640
NCH = N_EDGES // EB
WIN = 128
NPAD = 10112

_NC = 2
_NS = 16
_NW = _NC * _NS
_PER_TILE = N_EDGES // _NW
_CK = 128
_NFULL = _PER_TILE // _CK
_TAIL = _PER_TILE - _NFULL * _CK


def _sc_body(feat_hbm, ns_hbm, nt_hbm, perm_hbm, srcp_hbm, dsts_hbm,
             feat_out, gsrc_out, gdst_out,
             ixf, ixs, ixd, fb, sb, db, fbt, sbt, dbt,
             sem1, sem2, sem3):
    wid = lax.axis_index("s") * _NC + lax.axis_index("c")
    base = wid * _PER_TILE
    pltpu.sync_copy(perm_hbm.at[pl.ds(base, _PER_TILE)], ixf)
    pltpu.sync_copy(srcp_hbm.at[pl.ds(base, _PER_TILE)], ixs)
    pltpu.sync_copy(dsts_hbm.at[pl.ds(base, _PER_TILE)], ixd)

    def chunk(loc, fbuf, sbuf, dbuf, k):
        off = base + loc
        a = pltpu.async_copy(feat_hbm.at[ixf.at[pl.ds(loc, k)]], fbuf, sem1)
        b = pltpu.async_copy(ns_hbm.at[ixs.at[pl.ds(loc, k)]], sbuf, sem2)
        c = pltpu.async_copy(nt_hbm.at[ixd.at[pl.ds(loc, k)]], dbuf, sem3)
        a.wait()
        pltpu.sync_copy(fbuf, feat_out.at[pl.ds(off, k)])
        b.wait()
        pltpu.sync_copy(sbuf, gsrc_out.at[pl.ds(off, k)])
        c.wait()
        pltpu.sync_copy(dbuf, gdst_out.at[pl.ds(off, k)])

    def body(i, carry):
        chunk(i * _CK, fb, sb, db, _CK)
        return carry

    lax.fori_loop(0, _NFULL, body, 0)
    chunk(_NFULL * _CK, fbt, sbt, dbt, _TAIL)


def _sc_gather(feat, ns, nt, perm, src_p, dst_s):
    mesh = plsc.VectorSubcoreMesh(core_axis_name="c", subcore_axis_name="s")
    f = functools.partial(
        pl.kernel, mesh=mesh,
        out_type=[
            jax.ShapeDtypeStruct((N_EDGES, FEATC), jnp.float32),
            jax.ShapeDtypeStruct((N_EDGES, HIDDEN), jnp.float32),
            jax.ShapeDtypeStruct((N_EDGES, HIDDEN), jnp.float32),
        ],
        scratch_types=[
            pltpu.VMEM((_PER_TILE,), jnp.int32),
            pltpu.VMEM((_PER_TILE,), jnp.int32),
            pltpu.VMEM((_PER_TILE,), jnp.int32),
            pltpu.VMEM((_CK, FEATC), jnp.float32),
            pltpu.VMEM((_CK, HIDDEN), jnp.float32),
            pltpu.VMEM((_CK, HIDDEN), jnp.float32),
            pltpu.VMEM((_TAIL, FEATC), jnp.float32),
            pltpu.VMEM((_TAIL, HIDDEN), jnp.float32),
            pltpu.VMEM((_TAIL, HIDDEN), jnp.float32),
            pltpu.SemaphoreType.DMA,
            pltpu.SemaphoreType.DMA,
            pltpu.SemaphoreType.DMA,
        ],
    )
    return f(_sc_body)(feat, ns, nt, perm, src_p, dst_s)


def _tc_body(win_lo_ref, n_win_ref, dst_col_ref,
             feat_ref, gsrc_ref, gdst_ref, w1d_ref, w2_ref,
             b2_ref, out_ref, coeff_ref):
    b = pl.program_id(0)

    @pl.when(b == 0)
    def _init():
        out_ref[...] = jnp.zeros_like(out_ref)

    gsum = (gsrc_ref[...] + gdst_ref[...]).astype(jnp.bfloat16)
    r128 = jax.lax.broadcasted_iota(jnp.int32, (HIDDEN, HIDDEN), 0)
    c128 = jax.lax.broadcasted_iota(jnp.int32, (HIDDEN, HIDDEN), 1)
    eye128 = (r128 == c128).astype(jnp.bfloat16)
    dist = feat_ref[:, :NUM_RBF].astype(jnp.bfloat16)
    c11 = (((1,), (1,)), ((), ()))
    x = lax.dot_general(w1d_ref[...], dist, c11,
                        preferred_element_type=jnp.float32)
    x = x + lax.dot_general(eye128, gsum, c11,
                            preferred_element_type=jnp.float32)
    h = jnp.maximum(x, 0.0).astype(jnp.bfloat16)
    m0 = jnp.dot(w2_ref[...], h, preferred_element_type=jnp.float32)
    m0 = (m0 + b2_ref[...]).astype(jnp.bfloat16)

    eye64 = eye128[:SPH, :SPH]
    wig = lax.dot_general(eye64, feat_ref[:, NUM_RBF:].astype(jnp.bfloat16),
                          c11, preferred_element_type=jnp.float32
                          ).astype(jnp.bfloat16)
    for j in range(M0C):
        m0j = m0[j * SPH:(j + 1) * SPH, :]
        for i in range(NUM_COEFF):
            r = i * M0C + j
            w_row = lax.broadcast_in_dim(wig[r:r + 1, :], (SPH, EB), (0, 1))
            contrib = w_row * m0j
            sl = slice(i * SPH, (i + 1) * SPH)
            if j == 0:
                coeff_ref[sl, :] = contrib
            else:
                coeff_ref[sl, :] += contrib

    dst_col = dst_col_ref[0]
    win_lo = win_lo_ref[b]
    n_win = n_win_ref[b]
    coeff = coeff_ref[...]

    def body(w, carry):
        base = (win_lo + w) * WIN
        lane = jax.lax.broadcasted_iota(jnp.int32, (EB, WIN), 1).astype(jnp.float32)
        oh = (lane == (dst_col - base.astype(jnp.float32))).astype(jnp.bfloat16)
        contrib = jnp.dot(coeff, oh, preferred_element_type=jnp.float32)
        out_ref[:, pl.ds(base, WIN)] += contrib
        return carry

    lax.fori_loop(0, n_win, body, 0)


def kernel(atomic_numbers, edge_distance, edge_index, wigner_inv,
           source_embedding, target_embedding, W1, b1, W2, b2):
    src = edge_index[0]
    dst = edge_index[1]
    eid = jnp.arange(N_EDGES, dtype=jnp.uint32)
    key = dst.astype(jnp.uint32) * jnp.uint32(1 << 18) + eid
    val = src.astype(jnp.uint32) * jnp.uint32(1 << 18) + eid
    key_s, val_s = lax.sort([key, val], num_keys=1)
    perm = (val_s & jnp.uint32((1 << 18) - 1)).astype(jnp.int32)
    src_p = (val_s >> 18).astype(jnp.int32)
    dst_s = (key_s >> 18).astype(jnp.int32)

    feat = jnp.concatenate(
        [edge_distance, wigner_inv.reshape(N_EDGES, NUM_COEFF * M0C)], axis=1)

    oh_elem = (atomic_numbers[:, None] ==
               jnp.arange(MAX_ELEM)[None, :]).astype(jnp.float32)
    ns = oh_elem @ (source_embedding @ W1[NUM_RBF:NUM_RBF + EDGE_CH]) + 0.5 * b1
    nt = oh_elem @ (target_embedding @ W1[NUM_RBF + EDGE_CH:]) + 0.5 * b1

    feat_srt, gsrc, gdst = _sc_gather(feat, ns, nt, perm, src_p, dst_s)

    w1d_t = W1[:NUM_RBF].T.astype(jnp.bfloat16)
    w2_t = (W2 / RESCALE).T.astype(jnp.bfloat16)
    b2_c = (b2 / RESCALE).reshape(-1, 1)

    win_lo = (dst_s[::EB] // WIN).astype(jnp.int32)
    win_hi = (dst_s[EB - 1::EB] // WIN).astype(jnp.int32)
    n_win = win_hi - win_lo + 1

    dst_col = dst_s.astype(jnp.float32).reshape(NCH, EB, 1)

    grid_spec = pltpu.PrefetchScalarGridSpec(
        num_scalar_prefetch=2,
        grid=(NCH,),
        in_specs=[
            pl.BlockSpec((1, EB, 1), lambda b, *_: (b, 0, 0)),
            pl.BlockSpec((EB, FEATC), lambda b, *_: (b, 0)),
            pl.BlockSpec((EB, HIDDEN), lambda b, *_: (b, 0)),
            pl.BlockSpec((EB, HIDDEN), lambda b, *_: (b, 0)),
            pl.BlockSpec((HIDDEN, NUM_RBF), lambda b, *_: (0, 0)),
            pl.BlockSpec((M0C * SPH, HIDDEN), lambda b, *_: (0, 0)),
            pl.BlockSpec((M0C * SPH, 1), lambda b, *_: (0, 0)),
        ],
        out_specs=pl.BlockSpec((OUT_CH, NPAD), lambda b, *_: (0, 0)),
        scratch_shapes=[pltpu.VMEM((OUT_CH, EB), jnp.bfloat16)],
    )
    out = pl.pallas_call(
        _tc_body,
        grid_spec=grid_spec,
        out_shape=jax.ShapeDtypeStruct((OUT_CH, NPAD), jnp.float32),
        compiler_params=pltpu.CompilerParams(
            dimension_semantics=("arbitrary",)),
    )(win_lo, n_win, dst_col, feat_srt, gsrc, gdst, w1d_t, w2_t, b2_c)
    return out[:, :N_NODES].T.reshape(N_NODES, NUM_COEFF, SPH)
